# baseline (speedup 1.0000x reference)
import jax
import jax.numpy as jnp
from jax import lax
from jax.experimental import pallas as pl
from jax.experimental.pallas import tpu as pltpu
from jax.experimental.pallas import tpu_sc as plsc

N_IDENT = 10000
N_SYM = 10000
N_AST = 100000
N_OCC = 320000
D = 128

NC, NS = 2, 16
NW = NC * NS
CHUNK = 128
OCC_W = 10240
NCHUNK = OCC_W // CHUNK
OCC_RW = N_OCC // NW
SYM_W = 384
SYM_PAD = NW * SYM_W
SYM_CHUNKS = SYM_W // CHUNK
SEG_PAD = 10240
ROWS_T = SEG_PAD // NS
NCP = ROWS_T // CHUNK


def _sc_gather_segsum(ident_tab, sym_idx, ast_tab, occ_idx):
  mesh = plsc.VectorSubcoreMesh(
      core_axis_name="c", subcore_axis_name="s", num_cores=NC, num_subcores=NS)

  def body(ident_hbm, sym_hbm, ast_hbm, oidx_hbm, a_out, b_out,
           symv, cidx, rows_a, rows_b, acc, sem_a, sem_b, sem_i):
    c = lax.axis_index("c")
    s = lax.axis_index("s")
    wid = s * NC + c

    pltpu.sync_copy(sym_hbm.at[wid], symv)

    bufs = ((rows_a, sem_a), (rows_b, sem_b))
    for k in range(SYM_CHUNKS):
      buf, sem = bufs[k % 2]
      if k < 2:
        pltpu.async_copy(ident_hbm.at[symv.at[k]], buf, sem)
      pltpu.make_async_copy(ident_hbm.at[symv.at[k]], buf, sem).wait()
      pltpu.sync_copy(buf, a_out.at[wid, pl.ds(k * CHUNK, CHUNK)])
      if k + 2 < SYM_CHUNKS:
        pltpu.async_copy(ident_hbm.at[symv.at[k + 2]], buf, sem)

    zero = jnp.zeros((16,), jnp.float32)

    @pl.loop(0, CHUNK)
    def _zero_rows(i):
      for j in range(D // 16):
        rows_a[i, pl.ds(j * 16, 16)] = zero

    for m in range(NCP):
      pltpu.sync_copy(rows_a, acc.at[pl.ds(s * ROWS_T + m * CHUNK, CHUNK)])

    def stage_idx(j, slot, sync=False):
      if sync:
        pltpu.sync_copy(oidx_hbm.at[wid, j], cidx.at[slot])
      else:
        pltpu.async_copy(oidx_hbm.at[wid, j], cidx.at[slot], sem_i)

    def wait_stage(j, slot):
      pltpu.make_async_copy(oidx_hbm.at[wid, j], cidx.at[slot], sem_i).wait()

    def start_gather(slot, buf, sem):
      pltpu.async_copy(ast_hbm.at[cidx.at[slot, 0]], buf, sem)

    def wait_gather(slot, buf, sem):
      pltpu.make_async_copy(ast_hbm.at[cidx.at[slot, 0]], buf, sem).wait()

    stage_idx(0, 0, sync=True)
    stage_idx(1, 1, sync=True)
    stage_idx(2, 2)
    stage_idx(3, 3)
    start_gather(0, rows_a, sem_a)
    start_gather(1, rows_b, sem_b)
    plsc.subcore_barrier()

    @pl.loop(0, NCHUNK, step=4)
    def _quad(j):
      for u in range(4):
        slot = u
        buf, sem = bufs[u % 2]
        wait_gather(slot, buf, sem)
        pltpu.sync_copy(buf, acc.at[cidx.at[slot, 1]], add=True)

        @pl.when(j + u + 4 < NCHUNK)
        def _():
          stage_idx(j + u + 4, slot)

        @pl.when(j + u + 2 < NCHUNK)
        def _():
          wait_stage(j + u + 2, (u + 2) % 4)
          start_gather((u + 2) % 4, buf, sem)

    plsc.subcore_barrier()

    for m in range(NCP):
      r0 = s * ROWS_T + m * CHUNK
      pltpu.sync_copy(acc.at[pl.ds(r0, CHUNK)], rows_a)
      pltpu.sync_copy(rows_a, b_out.at[c, pl.ds(r0, CHUNK)])

  f = pl.kernel(
      body,
      out_type=(
          jax.ShapeDtypeStruct((NW, SYM_W, D), jnp.float32),
          jax.ShapeDtypeStruct((NC, SEG_PAD, D), jnp.float32),
      ),
      mesh=mesh,
      scratch_types=(
          pltpu.VMEM((SYM_CHUNKS, CHUNK), jnp.int32),
          pltpu.VMEM((4, 2, CHUNK), jnp.int32),
          pltpu.VMEM((CHUNK, D), jnp.float32),
          pltpu.VMEM((CHUNK, D), jnp.float32),
          pltpu.VMEM_SHARED((SEG_PAD, D), jnp.float32),
          pltpu.SemaphoreType.DMA,
          pltpu.SemaphoreType.DMA,
          pltpu.SemaphoreType.DMA,
      ),
  )
  return f(ident_tab, sym_idx, ast_tab, occ_idx)


BLK = 1000


def _tc_combine(a, b_partial, w1, w2):
  def body(a_ref, b_ref, w1_ref, w2_ref, o_ref):
    acc = jnp.dot(a_ref[...], w1_ref[...],
                  preferred_element_type=jnp.float32,
                  precision=lax.Precision.HIGHEST)
    acc = acc + jnp.dot(b_ref[0] + b_ref[1], w2_ref[...],
                        preferred_element_type=jnp.float32,
                        precision=lax.Precision.HIGHEST)
    o_ref[...] = jnp.maximum(acc, 0.0)

  return pl.pallas_call(
      body,
      grid=(N_SYM // BLK,),
      in_specs=[
          pl.BlockSpec((BLK, D), lambda i: (i, 0)),
          pl.BlockSpec((NC, BLK, D), lambda i: (0, i, 0)),
          pl.BlockSpec((D, D), lambda i: (0, 0)),
          pl.BlockSpec((D, D), lambda i: (0, 0)),
      ],
      out_specs=pl.BlockSpec((BLK, D), lambda i: (i, 0)),
      out_shape=jax.ShapeDtypeStruct((N_SYM, D), jnp.float32),
  )(a, b_partial, w1, w2)


def kernel(encoded_identifiers, symbols_identifier_indices, encoded_ast_nodes,
           ast_nodes_with_symbol_leaf_nodes_indices,
           ast_nodes_with_symbol_leaf_symbol_idx, W):
  sym_idx = symbols_identifier_indices.astype(jnp.int32)
  sym_idx = jnp.concatenate(
      [sym_idx, jnp.zeros((SYM_PAD - N_SYM,), jnp.int32)]
  ).reshape(NW, SYM_CHUNKS, CHUNK)

  node_idx = ast_nodes_with_symbol_leaf_nodes_indices.astype(jnp.int32)
  node_idx = node_idx.reshape(NW, OCC_RW)
  seg_idx = ast_nodes_with_symbol_leaf_symbol_idx.astype(jnp.int32)
  seg_idx = seg_idx.reshape(NW, OCC_RW)
  npad = OCC_W - OCC_RW
  node_idx = jnp.concatenate(
      [node_idx, jnp.zeros((NW, npad), jnp.int32)], axis=1)
  dummy = N_SYM + jnp.arange(NW, dtype=jnp.int32)[:, None]
  seg_idx = jnp.concatenate(
      [seg_idx, jnp.broadcast_to(dummy, (NW, npad))], axis=1)
  occ_idx = jnp.stack([node_idx.reshape(NW, NCHUNK, CHUNK),
                       seg_idx.reshape(NW, NCHUNK, CHUNK)], axis=2)

  a_gath, b_partial = _sc_gather_segsum(
      encoded_identifiers, sym_idx, encoded_ast_nodes, occ_idx)
  a = a_gath.reshape(SYM_PAD, D)[:N_SYM]
  return _tc_combine(a, b_partial[:, :N_SYM], W[:D], W[D:])


# speedup vs baseline: 1.1494x; 1.1494x over previous
import jax
import jax.numpy as jnp
from jax import lax
from jax.experimental import pallas as pl
from jax.experimental.pallas import tpu as pltpu
from jax.experimental.pallas import tpu_sc as plsc

N_IDENT = 10000
N_SYM = 10000
N_AST = 100000
N_OCC = 320000
D = 128

NC, NS = 2, 16
NW = NC * NS
CHUNK = 80
OCC_W = 10240
NCHUNK = OCC_W // CHUNK
OCC_RW = N_OCC // NW
SYM_W = 320
SYM_PAD = NW * SYM_W
SYM_CHUNKS = SYM_W // CHUNK
SEG_PAD = 10240
ROWS_T = SEG_PAD // NS
NCP = ROWS_T // CHUNK


def _sc_gather_segsum(ident_tab, sym_idx, ast_tab, occ_idx):
  mesh = plsc.VectorSubcoreMesh(
      core_axis_name="c", subcore_axis_name="s", num_cores=NC, num_subcores=NS)

  def body(ident_hbm, sym_hbm, ast_hbm, oidx_hbm, a_out, b_out,
           symv, cidx, rows_a, rows_b, acc, sem_a, sem_b, sem_i):
    c = lax.axis_index("c")
    s = lax.axis_index("s")
    wid = s * NC + c

    pltpu.sync_copy(sym_hbm.at[wid], symv)

    bufs = ((rows_a, sem_a), (rows_b, sem_b))
    for k in range(SYM_CHUNKS):
      buf, sem = bufs[k % 2]
      if k < 2:
        pltpu.async_copy(ident_hbm.at[symv.at[k]], buf, sem)
      pltpu.make_async_copy(ident_hbm.at[symv.at[k]], buf, sem).wait()
      pltpu.sync_copy(buf, a_out.at[wid, pl.ds(k * CHUNK, CHUNK)])
      if k + 2 < SYM_CHUNKS:
        pltpu.async_copy(ident_hbm.at[symv.at[k + 2]], buf, sem)

    zero = jnp.zeros((16,), jnp.float32)

    @pl.loop(0, CHUNK)
    def _zero_rows(i):
      for j in range(D // 16):
        rows_a[i, pl.ds(j * 16, 16)] = zero

    for m in range(NCP):
      pltpu.sync_copy(rows_a, acc.at[pl.ds(s * ROWS_T + m * CHUNK, CHUNK)])

    def stage_idx(j, slot, sync=False):
      if sync:
        pltpu.sync_copy(oidx_hbm.at[wid, j], cidx.at[slot])
      else:
        pltpu.async_copy(oidx_hbm.at[wid, j], cidx.at[slot], sem_i)

    def wait_stage(j, slot):
      pltpu.make_async_copy(oidx_hbm.at[wid, j], cidx.at[slot], sem_i).wait()

    def start_gather(slot, buf, sem):
      pltpu.async_copy(ast_hbm.at[cidx.at[slot, 0]], buf, sem)

    def wait_gather(slot, buf, sem):
      pltpu.make_async_copy(ast_hbm.at[cidx.at[slot, 0]], buf, sem).wait()

    stage_idx(0, 0, sync=True)
    stage_idx(1, 1, sync=True)
    stage_idx(2, 2)
    stage_idx(3, 3)
    start_gather(0, rows_a, sem_a)
    start_gather(1, rows_b, sem_b)
    plsc.subcore_barrier()

    @pl.loop(0, NCHUNK, step=4)
    def _quad(j):
      for u in range(4):
        slot = u
        buf, sem = bufs[u % 2]
        wait_gather(slot, buf, sem)
        pltpu.sync_copy(buf, acc.at[cidx.at[slot, 1]], add=True)

        @pl.when(j + u + 4 < NCHUNK)
        def _():
          stage_idx(j + u + 4, slot)

        @pl.when(j + u + 2 < NCHUNK)
        def _():
          wait_stage(j + u + 2, (u + 2) % 4)
          start_gather((u + 2) % 4, buf, sem)

    plsc.subcore_barrier()

    for m in range(NCP):
      r0 = s * ROWS_T + m * CHUNK
      pltpu.sync_copy(acc.at[pl.ds(r0, CHUNK)], rows_a)
      pltpu.sync_copy(rows_a, b_out.at[c, pl.ds(r0, CHUNK)])

  f = pl.kernel(
      body,
      out_type=(
          jax.ShapeDtypeStruct((NW, SYM_W, D), jnp.float32),
          jax.ShapeDtypeStruct((NC, SEG_PAD, D), jnp.float32),
      ),
      mesh=mesh,
      scratch_types=(
          pltpu.VMEM((SYM_CHUNKS, CHUNK), jnp.int32),
          pltpu.VMEM((4, 2, CHUNK), jnp.int32),
          pltpu.VMEM((CHUNK, D), jnp.float32),
          pltpu.VMEM((CHUNK, D), jnp.float32),
          pltpu.VMEM_SHARED((SEG_PAD, D), jnp.float32),
          pltpu.SemaphoreType.DMA,
          pltpu.SemaphoreType.DMA,
          pltpu.SemaphoreType.DMA,
      ),
  )
  return f(ident_tab, sym_idx, ast_tab, occ_idx)


BLK = 1000


def _tc_combine(a, b_partial, w1, w2):
  def body(a_ref, b_ref, w1_ref, w2_ref, o_ref):
    acc = jnp.dot(a_ref[...], w1_ref[...],
                  preferred_element_type=jnp.float32,
                  precision=lax.Precision.HIGHEST)
    acc = acc + jnp.dot(b_ref[0] + b_ref[1], w2_ref[...],
                        preferred_element_type=jnp.float32,
                        precision=lax.Precision.HIGHEST)
    o_ref[...] = jnp.maximum(acc, 0.0)

  return pl.pallas_call(
      body,
      grid=(N_SYM // BLK,),
      in_specs=[
          pl.BlockSpec((BLK, D), lambda i: (i, 0)),
          pl.BlockSpec((NC, BLK, D), lambda i: (0, i, 0)),
          pl.BlockSpec((D, D), lambda i: (0, 0)),
          pl.BlockSpec((D, D), lambda i: (0, 0)),
      ],
      out_specs=pl.BlockSpec((BLK, D), lambda i: (i, 0)),
      out_shape=jax.ShapeDtypeStruct((N_SYM, D), jnp.float32),
  )(a, b_partial, w1, w2)


def kernel(encoded_identifiers, symbols_identifier_indices, encoded_ast_nodes,
           ast_nodes_with_symbol_leaf_nodes_indices,
           ast_nodes_with_symbol_leaf_symbol_idx, W):
  sym_idx = symbols_identifier_indices.astype(jnp.int32)
  sym_idx = jnp.concatenate(
      [sym_idx, jnp.zeros((SYM_PAD - N_SYM,), jnp.int32)]
  ).reshape(NW, SYM_CHUNKS, CHUNK)

  node_idx = ast_nodes_with_symbol_leaf_nodes_indices.astype(jnp.int32)
  node_idx = node_idx.reshape(NW, OCC_RW)
  seg_idx = ast_nodes_with_symbol_leaf_symbol_idx.astype(jnp.int32)
  seg_idx = seg_idx.reshape(NW, OCC_RW)
  npad = OCC_W - OCC_RW
  node_idx = jnp.concatenate(
      [node_idx, jnp.zeros((NW, npad), jnp.int32)], axis=1)
  dummy = N_SYM + jnp.arange(NW, dtype=jnp.int32)[:, None]
  seg_idx = jnp.concatenate(
      [seg_idx, jnp.broadcast_to(dummy, (NW, npad))], axis=1)
  occ_idx = jnp.stack([node_idx.reshape(NW, NCHUNK, CHUNK),
                       seg_idx.reshape(NW, NCHUNK, CHUNK)], axis=2)

  a_gath, b_partial = _sc_gather_segsum(
      encoded_identifiers, sym_idx, encoded_ast_nodes, occ_idx)
  a = a_gath.reshape(SYM_PAD, D)[:N_SYM]
  return _tc_combine(a, b_partial[:, :N_SYM], W[:D], W[D:])


# speedup vs baseline: 1.2420x; 1.0806x over previous
import jax
import jax.numpy as jnp
from jax import lax
from jax.experimental import pallas as pl
from jax.experimental.pallas import tpu as pltpu
from jax.experimental.pallas import tpu_sc as plsc

N_IDENT = 10000
N_SYM = 10000
N_AST = 100000
N_OCC = 320000
D = 128

NC, NS = 2, 16
NW = NC * NS
CHUNK = 80
OCC_W = 10240
NCHUNK = OCC_W // CHUNK
OCC_RW = N_OCC // NW
SYM_W = 320
SYM_PAD = NW * SYM_W
SYM_CHUNKS = SYM_W // CHUNK
SEG_PAD = 10240
ROWS_T = SEG_PAD // NS
NCP = ROWS_T // CHUNK


def _sc_gather_segsum(ident_tab, sym_idx, ast_tab, occ_idx):
  mesh = plsc.VectorSubcoreMesh(
      core_axis_name="c", subcore_axis_name="s", num_cores=NC, num_subcores=NS)

  def body(ident_hbm, sym_hbm, ast_hbm, oidx_hbm, a_out, b_out,
           symv, cidx, rows_a, rows_b, acc, sem_a, sem_b, sem_i):
    c = lax.axis_index("c")
    s = lax.axis_index("s")
    wid = s * NC + c

    pltpu.sync_copy(sym_hbm.at[wid], symv)

    bufs = ((rows_a, sem_a), (rows_b, sem_b))
    for k in range(SYM_CHUNKS):
      buf, sem = bufs[k % 2]
      if k < 2:
        pltpu.async_copy(ident_hbm.at[symv.at[k]], buf, sem)
      pltpu.make_async_copy(ident_hbm.at[symv.at[k]], buf, sem).wait()
      pltpu.sync_copy(buf, a_out.at[wid, pl.ds(k * CHUNK, CHUNK)])
      if k + 2 < SYM_CHUNKS:
        pltpu.async_copy(ident_hbm.at[symv.at[k + 2]], buf, sem)

    zero = jnp.zeros((16,), jnp.float32)

    @pl.loop(0, CHUNK)
    def _zero_rows(i):
      for j in range(D // 16):
        rows_a[i, pl.ds(j * 16, 16)] = zero

    for m in range(NCP):
      pltpu.sync_copy(rows_a, acc.at[pl.ds(s * ROWS_T + m * CHUNK, CHUNK)])

    def stage_idx(j, slot, sync=False):
      if sync:
        pltpu.sync_copy(oidx_hbm.at[wid, j], cidx.at[slot])
      else:
        pltpu.async_copy(oidx_hbm.at[wid, j], cidx.at[slot], sem_i)

    def wait_stage(j, slot):
      pltpu.make_async_copy(oidx_hbm.at[wid, j], cidx.at[slot], sem_i).wait()

    def start_gather(slot, buf, sem):
      pltpu.async_copy(ast_hbm.at[cidx.at[slot, 0]], buf, sem)

    def wait_gather(slot, buf, sem):
      pltpu.make_async_copy(ast_hbm.at[cidx.at[slot, 0]], buf, sem).wait()

    stage_idx(0, 0, sync=True)
    stage_idx(1, 1, sync=True)
    stage_idx(2, 2)
    stage_idx(3, 3)
    start_gather(0, rows_a, sem_a)
    start_gather(1, rows_b, sem_b)
    plsc.subcore_barrier()

    @pl.loop(0, NCHUNK, step=4)
    def _quad(j):
      for u in range(4):
        slot = u
        buf, sem = bufs[u % 2]
        wait_gather(slot, buf, sem)
        pltpu.sync_copy(buf, acc.at[cidx.at[slot, 1]], add=True)

        @pl.when(j + u + 4 < NCHUNK)
        def _():
          stage_idx(j + u + 4, slot)

        @pl.when(j + u + 2 < NCHUNK)
        def _():
          wait_stage(j + u + 2, (u + 2) % 4)
          start_gather((u + 2) % 4, buf, sem)

    plsc.subcore_barrier()

    for m in range(NCP):
      r0 = s * ROWS_T + m * CHUNK
      pltpu.sync_copy(acc.at[pl.ds(r0, CHUNK)], rows_a)
      pltpu.sync_copy(rows_a, b_out.at[c, pl.ds(r0, CHUNK)])

  f = pl.kernel(
      body,
      out_type=(
          jax.ShapeDtypeStruct((NW, SYM_W, D), jnp.float32),
          jax.ShapeDtypeStruct((NC, SEG_PAD, D), jnp.float32),
      ),
      mesh=mesh,
      scratch_types=(
          pltpu.VMEM((SYM_CHUNKS, CHUNK), jnp.int32),
          pltpu.VMEM((4, 2, CHUNK), jnp.int32),
          pltpu.VMEM((CHUNK, D), jnp.float32),
          pltpu.VMEM((CHUNK, D), jnp.float32),
          pltpu.VMEM_SHARED((SEG_PAD, D), jnp.float32),
          pltpu.SemaphoreType.DMA,
          pltpu.SemaphoreType.DMA,
          pltpu.SemaphoreType.DMA,
      ),
  )
  return f(ident_tab, sym_idx, ast_tab, occ_idx)


BLK = 1000


def _tc_combine(a, b_partial, w1, w2):
  def body(a_ref, b_ref, w1_ref, w2_ref, o_ref):
    acc = jnp.dot(a_ref[...], w1_ref[...],
                  preferred_element_type=jnp.float32,
                  precision=lax.Precision.HIGHEST)
    acc = acc + jnp.dot(b_ref[0] + b_ref[1], w2_ref[...],
                        preferred_element_type=jnp.float32,
                        precision=lax.Precision.HIGHEST)
    o_ref[...] = jnp.maximum(acc, 0.0)

  return pl.pallas_call(
      body,
      grid=(N_SYM // BLK,),
      in_specs=[
          pl.BlockSpec((BLK, D), lambda i: (i, 0)),
          pl.BlockSpec((NC, BLK, D), lambda i: (0, i, 0)),
          pl.BlockSpec((D, D), lambda i: (0, 0)),
          pl.BlockSpec((D, D), lambda i: (0, 0)),
      ],
      out_specs=pl.BlockSpec((BLK, D), lambda i: (i, 0)),
      out_shape=jax.ShapeDtypeStruct((N_SYM, D), jnp.float32),
  )(a, b_partial, w1, w2)


def kernel(encoded_identifiers, symbols_identifier_indices, encoded_ast_nodes,
           ast_nodes_with_symbol_leaf_nodes_indices,
           ast_nodes_with_symbol_leaf_symbol_idx, W):
  sym_idx = symbols_identifier_indices.astype(jnp.int32)
  sym_idx = jnp.concatenate(
      [sym_idx, jnp.zeros((SYM_PAD - N_SYM,), jnp.int32)]
  ).reshape(NW, SYM_CHUNKS, CHUNK)

  node_idx = ast_nodes_with_symbol_leaf_nodes_indices.astype(jnp.int32)
  node_idx = node_idx.reshape(NW, OCC_RW)
  seg_idx = ast_nodes_with_symbol_leaf_symbol_idx.astype(jnp.int32)
  seg_idx = seg_idx.reshape(NW, OCC_RW)
  npad = OCC_W - OCC_RW
  node_idx = jnp.concatenate(
      [node_idx, jnp.zeros((NW, npad), jnp.int32)], axis=1)
  dummy = N_SYM + jnp.arange(NW, dtype=jnp.int32)[:, None]
  seg_idx = jnp.concatenate(
      [seg_idx, jnp.broadcast_to(dummy, (NW, npad))], axis=1)
  node_idx = node_idx.reshape(NW, CHUNK, NCHUNK).transpose(0, 2, 1)
  seg_idx = seg_idx.reshape(NW, CHUNK, NCHUNK).transpose(0, 2, 1)
  occ_idx = jnp.stack([node_idx, seg_idx], axis=2)

  a_gath, b_partial = _sc_gather_segsum(
      encoded_identifiers, sym_idx, encoded_ast_nodes, occ_idx)
  a = a_gath.reshape(SYM_PAD, D)[:N_SYM]
  return _tc_combine(a, b_partial[:, :N_SYM], W[:D], W[D:])
